# Initial kernel scaffold; baseline (speedup 1.0000x reference)
import jax
import jax.numpy as jnp
from jax.experimental import pallas as pl


def kernel(adj_indices, embeds, qTrans, kTrans, vTrans):
    raise NotImplementedError("write your pallas kernel here")


# speedup vs baseline: 1.4746x; 1.4746x over previous
import functools

import jax
import jax.numpy as jnp
from jax import lax
from jax.experimental import pallas as pl
from jax.experimental.pallas import tpu as pltpu
from jax.experimental.pallas import tpu_sc as plsc

LATDIM = 256
HEAD = 4
DH = LATDIM // HEAD
N = 10000
E = 160000
NP = 10240
NF = NP * HEAD
NC = 2
NS = 16
NW = NC * NS
CE = 64
EWP = 5056
EP = EWP * NW
ECT = EP // NS
SL = NF // NS
HF = LATDIM // 2
RT = NP // NS

_mesh = plsc.VectorSubcoreMesh(core_axis_name="c", subcore_axis_name="s")


def _qkv_body(x_ref, q_ref, k_ref, v_ref, oq_ref, ok_ref, ov_ref):
    x = x_ref[...]
    oq_ref[...] = jnp.dot(x, q_ref[...], preferred_element_type=jnp.float32)
    ok_ref[...] = jnp.dot(x, k_ref[...], preferred_element_type=jnp.float32)
    v = jnp.dot(x, v_ref[...], preferred_element_type=jnp.float32)
    ov_ref[0] = v[:, :HF]
    ov_ref[1] = v[:, HF:]


_qkv = pl.pallas_call(
    _qkv_body,
    grid=(NP // 1024,),
    in_specs=[
        pl.BlockSpec((1024, LATDIM), lambda i: (i, 0)),
        pl.BlockSpec((LATDIM, LATDIM), lambda i: (0, 0)),
        pl.BlockSpec((LATDIM, LATDIM), lambda i: (0, 0)),
        pl.BlockSpec((LATDIM, LATDIM), lambda i: (0, 0)),
    ],
    out_specs=[
        pl.BlockSpec((1024, LATDIM), lambda i: (i, 0)),
        pl.BlockSpec((1024, LATDIM), lambda i: (i, 0)),
        pl.BlockSpec((2, 1024, HF), lambda i: (0, i, 0)),
    ],
    out_shape=[
        jax.ShapeDtypeStruct((NP, LATDIM), jnp.float32),
        jax.ShapeDtypeStruct((NP, LATDIM), jnp.float32),
        jax.ShapeDtypeStruct((2, NP, HF), jnp.float32),
    ],
)


def _lane_total(acc, iot):
    for bit in (1, 2, 4, 8):
        acc = acc + jnp.take(acc, iot ^ bit, mode='fill')
    return acc


def _splats(v, iot):
    vs = [v]
    for bit in (8, 4, 2, 1):
        nxt = []
        for u in vs:
            p = jnp.take(u, iot ^ bit, mode='fill')
            keep = (iot & bit) == 0
            nxt.append(jnp.where(keep, u, p))
            nxt.append(jnp.where(keep, p, u))
        vs = nxt
    return vs


def _edge_body(rows_hbm, cols_hbm, q_hbm, k_hbm,
               exp_hbm, np0_hbm, np1_hbm,
               rowc, colc, qbuf, kbuf, evbuf, idxh, zbuf, normsp, sem):
    c = lax.axis_index("c")
    s = lax.axis_index("s")
    w = s * NC + c
    iot = lax.iota(jnp.int32, 16)
    zero16 = jnp.zeros((16,), jnp.float32)

    def zb(j, _):
        zbuf[pl.ds(j * 16, 16)] = zero16
        return 0
    lax.fori_loop(0, SL // 16, zb, 0)
    pltpu.sync_copy(zbuf, normsp.at[pl.ds(s * SL, SL)])
    plsc.subcore_barrier()

    def chunk(i, _):
        b = w * EWP + i * CE
        pltpu.sync_copy(rows_hbm.at[pl.ds(b, CE)], rowc)
        pltpu.sync_copy(cols_hbm.at[pl.ds(b, CE)], colc)
        pltpu.async_copy(q_hbm.at[rowc], qbuf, sem).wait()
        pltpu.async_copy(k_hbm.at[colc], kbuf, sem).wait()
        for h in range(HEAD):
            for m in range(CE // 16):
                idxh[h, pl.ds(m * 16, 16)] = (
                    rowc[pl.ds(m * 16, 16)] + h * NP)

        def grp(g, _):
            dph = [jnp.zeros((16,), jnp.float32) for _ in range(HEAD)]
            for t in range(16):
                e = g * 16 + t
                for h in range(HEAD):
                    acc = (qbuf[e, pl.ds(h * DH, 16)]
                           * kbuf[e, pl.ds(h * DH, 16)])
                    for j in range(1, 4):
                        o = h * DH + j * 16
                        acc = acc + (qbuf[e, pl.ds(o, 16)]
                                     * kbuf[e, pl.ds(o, 16)])
                    tot = _lane_total(acc, iot)
                    dph[h] = jnp.where(iot == t, tot, dph[h])
            for h in range(HEAD):
                v = jnp.minimum(jnp.maximum(dph[h], -10.0), 10.0)
                evbuf[h, pl.ds(g * 16, 16)] = jnp.exp(v)
            return 0
        lax.fori_loop(0, CE // 16, grp, 0)

        for h in range(HEAD):
            pltpu.sync_copy(evbuf.at[h], exp_hbm.at[pl.ds(h * EP + b, CE)])
            pltpu.sync_copy(evbuf.at[h], normsp.at[idxh.at[h]], add=True)
        return 0
    lax.fori_loop(0, EWP // CE, chunk, 0)

    plsc.subcore_barrier()

    @pl.when(c == 0)
    def _():
        pltpu.sync_copy(normsp.at[pl.ds(s * SL, SL)],
                        np0_hbm.at[pl.ds(s * SL, SL)])

    @pl.when(c == 1)
    def _():
        pltpu.sync_copy(normsp.at[pl.ds(s * SL, SL)],
                        np1_hbm.at[pl.ds(s * SL, SL)])


_edge_pass = functools.partial(
    pl.kernel,
    out_type=[
        jax.ShapeDtypeStruct((HEAD * EP,), jnp.float32),
        jax.ShapeDtypeStruct((NF,), jnp.float32),
        jax.ShapeDtypeStruct((NF,), jnp.float32),
    ],
    mesh=_mesh,
    scratch_types=[
        pltpu.VMEM((CE,), jnp.int32),
        pltpu.VMEM((CE,), jnp.int32),
        pltpu.VMEM((CE, LATDIM), jnp.float32),
        pltpu.VMEM((CE, LATDIM), jnp.float32),
        pltpu.VMEM((HEAD, CE), jnp.float32),
        pltpu.VMEM((HEAD, CE), jnp.int32),
        pltpu.VMEM((SL,), jnp.float32),
        pltpu.VMEM_SHARED((NF,), jnp.float32),
        pltpu.SemaphoreType.DMA,
    ],
)(_edge_body)


def _agg_body(rows_hbm, cols_hbm, exp_hbm, np0_hbm, np1_hbm, v_hbm,
              att_hbm, res_hbm,
              rowc, colc, idxh, expb, nb0, nb1, attb, m0b, m1b,
              vbuf, wbuf, nh0, nh1, ng0, ng1, accsp, sem):
    c = lax.axis_index("c")
    s = lax.axis_index("s")
    w = s * NC + c
    iot = lax.iota(jnp.int32, 16)
    zero16 = jnp.zeros((16,), jnp.float32)

    def zrow(r, _):
        for j in range(HF // 16):
            wbuf[r, pl.ds(j * 16, 16)] = zero16
        return 0
    lax.fori_loop(0, 128, zrow, 0)
    for q in range(RT // 128):
        pltpu.sync_copy(wbuf, accsp.at[pl.ds(s * RT + q * 128, 128)])

    def chunk1(i, _):
        b = w * EWP + i * CE
        pltpu.sync_copy(rows_hbm.at[pl.ds(b, CE)], rowc)
        for h in range(HEAD):
            for m in range(CE // 16):
                idxh[h, pl.ds(m * 16, 16)] = (
                    rowc[pl.ds(m * 16, 16)] + h * NP)
        for h in range(HEAD):
            pltpu.sync_copy(exp_hbm.at[pl.ds(h * EP + b, CE)], expb)
            pltpu.async_copy(np0_hbm.at[idxh.at[h]], nb0, sem).wait()
            pltpu.async_copy(np1_hbm.at[idxh.at[h]], nb1, sem).wait()
            for m in range(CE // 16):
                sl = pl.ds(m * 16, 16)
                attb[sl] = expb[sl] / (nb0[sl] + nb1[sl] + 1e-8)
            pltpu.sync_copy(attb, att_hbm.at[pl.ds(h * EP + b, CE)])
        return 0
    lax.fori_loop(0, EWP // CE, chunk1, 0)

    plsc.subcore_barrier()

    def chunk2(i, _):
        b = s * ECT + i * CE
        pltpu.sync_copy(rows_hbm.at[pl.ds(b, CE)], rowc)
        pltpu.sync_copy(cols_hbm.at[pl.ds(b, CE)], colc)
        for m in range(CE // 16):
            colc[pl.ds(m * 16, 16)] = colc[pl.ds(m * 16, 16)] + c * NP
        pltpu.async_copy(v_hbm.at[colc], vbuf, sem).wait()
        pltpu.sync_copy(exp_hbm.at[pl.ds((2 * c) * EP + b, CE)], m0b)
        pltpu.sync_copy(exp_hbm.at[pl.ds((2 * c + 1) * EP + b, CE)], m1b)

        def grp(g, _):
            s0 = _splats(m0b[pl.ds(g * 16, 16)], iot)
            s1 = _splats(m1b[pl.ds(g * 16, 16)], iot)
            for t in range(16):
                e = g * 16 + t
                for j in range(HF // 16):
                    bb = s0[t] if j < 4 else s1[t]
                    vbuf[e, pl.ds(j * 16, 16)] = (
                        vbuf[e, pl.ds(j * 16, 16)] * bb)
            return 0
        lax.fori_loop(0, CE // 16, grp, 0)
        pltpu.sync_copy(vbuf, accsp.at[rowc], add=True)
        return 0
    lax.fori_loop(0, ECT // CE, chunk2, 0)

    plsc.subcore_barrier()

    for q in range(RT // 128):
        r0 = s * RT + q * 128
        pltpu.sync_copy(accsp.at[pl.ds(r0, 128)], wbuf)
        pltpu.sync_copy(np0_hbm.at[pl.ds((2 * c) * NP + r0, 128)], nh0)
        pltpu.sync_copy(np1_hbm.at[pl.ds((2 * c) * NP + r0, 128)], ng0)
        pltpu.sync_copy(np0_hbm.at[pl.ds((2 * c + 1) * NP + r0, 128)], nh1)
        pltpu.sync_copy(np1_hbm.at[pl.ds((2 * c + 1) * NP + r0, 128)], ng1)
        for m in range(8):
            sl = pl.ds(m * 16, 16)
            nh0[sl] = 1.0 / (nh0[sl] + ng0[sl] + 1e-8)
            nh1[sl] = 1.0 / (nh1[sl] + ng1[sl] + 1e-8)

        def grp3(g, _):
            i0 = _splats(nh0[pl.ds(g * 16, 16)], iot)
            i1 = _splats(nh1[pl.ds(g * 16, 16)], iot)
            for t in range(16):
                r = g * 16 + t
                for j in range(HF // 16):
                    bb = i0[t] if j < 4 else i1[t]
                    wbuf[r, pl.ds(j * 16, 16)] = (
                        wbuf[r, pl.ds(j * 16, 16)] * bb)
            return 0
        lax.fori_loop(0, 8, grp3, 0)
        pltpu.sync_copy(wbuf, res_hbm.at[pl.ds(c * NP + r0, 128)])


_agg_pass = functools.partial(
    pl.kernel,
    out_type=[
        jax.ShapeDtypeStruct((HEAD * EP,), jnp.float32),
        jax.ShapeDtypeStruct((2 * NP, HF), jnp.float32),
    ],
    mesh=_mesh,
    scratch_types=[
        pltpu.VMEM((CE,), jnp.int32),
        pltpu.VMEM((CE,), jnp.int32),
        pltpu.VMEM((HEAD, CE), jnp.int32),
        pltpu.VMEM((CE,), jnp.float32),
        pltpu.VMEM((CE,), jnp.float32),
        pltpu.VMEM((CE,), jnp.float32),
        pltpu.VMEM((CE,), jnp.float32),
        pltpu.VMEM((CE,), jnp.float32),
        pltpu.VMEM((CE,), jnp.float32),
        pltpu.VMEM((CE, HF), jnp.float32),
        pltpu.VMEM((128, HF), jnp.float32),
        pltpu.VMEM((128,), jnp.float32),
        pltpu.VMEM((128,), jnp.float32),
        pltpu.VMEM((128,), jnp.float32),
        pltpu.VMEM((128,), jnp.float32),
        pltpu.VMEM_SHARED((NP, HF), jnp.float32),
        pltpu.SemaphoreType.DMA,
    ],
)(_agg_body)


def kernel(adj_indices, embeds, qTrans, kTrans, vTrans):
    rows = adj_indices[0]
    cols = adj_indices[1]
    pad_e = EP - E
    rows_p = jnp.concatenate(
        [rows, jnp.full((pad_e,), N + 200, jnp.int32)])
    cols_p = jnp.concatenate(
        [cols, jnp.full((pad_e,), N + 200, jnp.int32)])
    embeds_p = jnp.pad(embeds, ((0, NP - N), (0, 0)))
    q, k, vs = _qkv(embeds_p, qTrans, kTrans, vTrans)
    vflat = vs.reshape(2 * NP, HF)
    exp_flat, np0, np1 = _edge_pass(rows_p, cols_p, q, k)
    att_flat, res2 = _agg_pass(rows_p, cols_p, exp_flat, np0, np1, vflat)
    res = jnp.concatenate([res2[:N], res2[NP:NP + N]], axis=1)
    att = att_flat.reshape(HEAD, EP)[:, :E].T
    return res, att


# speedup vs baseline: 1.8595x; 1.2611x over previous
import functools

import jax
import jax.numpy as jnp
from jax import lax
from jax.experimental import pallas as pl
from jax.experimental.pallas import tpu as pltpu
from jax.experimental.pallas import tpu_sc as plsc

LATDIM = 256
HEAD = 4
DH = LATDIM // HEAD
N = 10000
E = 160000
NP = 10240
NF = NP * HEAD
NC = 2
NS = 16
NW = NC * NS
CE = 128
EWP = 5120
EP = EWP * NW
ECT = EP // NS
SL = NF // NS
HF = LATDIM // 2
RT = NP // NS

_mesh = plsc.VectorSubcoreMesh(core_axis_name="c", subcore_axis_name="s")


def _qkv_body(x_ref, q_ref, k_ref, v_ref, oq_ref, ok_ref, ov_ref):
    x = x_ref[...]
    oq_ref[...] = jnp.dot(x, q_ref[...], preferred_element_type=jnp.float32)
    ok_ref[...] = jnp.dot(x, k_ref[...], preferred_element_type=jnp.float32)
    v = jnp.dot(x, v_ref[...], preferred_element_type=jnp.float32)
    ov_ref[0] = v[:, :HF]
    ov_ref[1] = v[:, HF:]


_qkv = pl.pallas_call(
    _qkv_body,
    grid=(NP // 1024,),
    in_specs=[
        pl.BlockSpec((1024, LATDIM), lambda i: (i, 0)),
        pl.BlockSpec((LATDIM, LATDIM), lambda i: (0, 0)),
        pl.BlockSpec((LATDIM, LATDIM), lambda i: (0, 0)),
        pl.BlockSpec((LATDIM, LATDIM), lambda i: (0, 0)),
    ],
    out_specs=[
        pl.BlockSpec((1024, LATDIM), lambda i: (i, 0)),
        pl.BlockSpec((1024, LATDIM), lambda i: (i, 0)),
        pl.BlockSpec((2, 1024, HF), lambda i: (0, i, 0)),
    ],
    out_shape=[
        jax.ShapeDtypeStruct((NP, LATDIM), jnp.float32),
        jax.ShapeDtypeStruct((NP, LATDIM), jnp.float32),
        jax.ShapeDtypeStruct((2, NP, HF), jnp.float32),
    ],
)


def _lane_total(acc, iot):
    for bit in (1, 2, 4, 8):
        acc = acc + jnp.take(acc, iot ^ bit, mode='fill')
    return acc


def _splats(v, iot):
    vs = [v]
    for bit in (8, 4, 2, 1):
        nxt = []
        for u in vs:
            p = jnp.take(u, iot ^ bit, mode='fill')
            keep = (iot & bit) == 0
            nxt.append(jnp.where(keep, u, p))
            nxt.append(jnp.where(keep, p, u))
        vs = nxt
    return vs


def _edge_body(rows_hbm, cols_hbm, q_hbm, k_hbm,
               exp_hbm, np0_hbm, np1_hbm,
               rowc, colc, qbuf, kbuf, evbuf, idxh, zbuf, normsp, sem):
    c = lax.axis_index("c")
    s = lax.axis_index("s")
    w = s * NC + c
    iot = lax.iota(jnp.int32, 16)
    zero16 = jnp.zeros((16,), jnp.float32)

    def zb(j, _):
        zbuf[pl.ds(j * 16, 16)] = zero16
        return 0
    lax.fori_loop(0, SL // 16, zb, 0)
    pltpu.sync_copy(zbuf, normsp.at[pl.ds(s * SL, SL)])
    plsc.subcore_barrier()

    def chunk(i, _):
        b = w * EWP + i * CE
        pltpu.sync_copy(rows_hbm.at[pl.ds(b, CE)], rowc)
        pltpu.sync_copy(cols_hbm.at[pl.ds(b, CE)], colc)
        dq = pltpu.async_copy(q_hbm.at[rowc], qbuf, sem)
        dk = pltpu.async_copy(k_hbm.at[colc], kbuf, sem)
        dq.wait()
        dk.wait()
        for h in range(HEAD):
            for m in range(CE // 16):
                idxh[h, pl.ds(m * 16, 16)] = (
                    rowc[pl.ds(m * 16, 16)] + h * NP)

        def grp(g, _):
            dph = [jnp.zeros((16,), jnp.float32) for _ in range(HEAD)]
            for t in range(16):
                e = g * 16 + t
                for h in range(HEAD):
                    acc = (qbuf[e, pl.ds(h * DH, 16)]
                           * kbuf[e, pl.ds(h * DH, 16)])
                    for j in range(1, 4):
                        o = h * DH + j * 16
                        acc = acc + (qbuf[e, pl.ds(o, 16)]
                                     * kbuf[e, pl.ds(o, 16)])
                    tot = _lane_total(acc, iot)
                    dph[h] = jnp.where(iot == t, tot, dph[h])
            for h in range(HEAD):
                v = jnp.minimum(jnp.maximum(dph[h], -10.0), 10.0)
                evbuf[h, pl.ds(g * 16, 16)] = jnp.exp(v)
            return 0
        lax.fori_loop(0, CE // 16, grp, 0)

        for h in range(HEAD):
            pltpu.sync_copy(evbuf.at[h], exp_hbm.at[pl.ds(h * EP + b, CE)])
            pltpu.sync_copy(evbuf.at[h], normsp.at[idxh.at[h]], add=True)
        return 0
    lax.fori_loop(0, EWP // CE, chunk, 0)

    plsc.subcore_barrier()

    @pl.when(c == 0)
    def _():
        pltpu.sync_copy(normsp.at[pl.ds(s * SL, SL)],
                        np0_hbm.at[pl.ds(s * SL, SL)])

    @pl.when(c == 1)
    def _():
        pltpu.sync_copy(normsp.at[pl.ds(s * SL, SL)],
                        np1_hbm.at[pl.ds(s * SL, SL)])


_edge_pass = functools.partial(
    pl.kernel,
    out_type=[
        jax.ShapeDtypeStruct((HEAD * EP,), jnp.float32),
        jax.ShapeDtypeStruct((NF,), jnp.float32),
        jax.ShapeDtypeStruct((NF,), jnp.float32),
    ],
    mesh=_mesh,
    scratch_types=[
        pltpu.VMEM((CE,), jnp.int32),
        pltpu.VMEM((CE,), jnp.int32),
        pltpu.VMEM((CE, LATDIM), jnp.float32),
        pltpu.VMEM((CE, LATDIM), jnp.float32),
        pltpu.VMEM((HEAD, CE), jnp.float32),
        pltpu.VMEM((HEAD, CE), jnp.int32),
        pltpu.VMEM((SL,), jnp.float32),
        pltpu.VMEM_SHARED((NF,), jnp.float32),
        pltpu.SemaphoreType.DMA,
    ],
)(_edge_body)


def _agg_body(rows_hbm, cols_hbm, exp_hbm, np0_hbm, np1_hbm, v_hbm,
              att_hbm, res_hbm,
              rowc, colc, idxh, expb, nb0, nb1, attb, m0b, m1b,
              vbuf, wbuf, nh0, nh1, ng0, ng1, accsp, sem):
    c = lax.axis_index("c")
    s = lax.axis_index("s")
    w = s * NC + c
    iot = lax.iota(jnp.int32, 16)
    zero16 = jnp.zeros((16,), jnp.float32)

    def zrow(r, _):
        for j in range(HF // 16):
            wbuf[r, pl.ds(j * 16, 16)] = zero16
        return 0
    lax.fori_loop(0, 128, zrow, 0)
    for q in range(RT // 128):
        pltpu.sync_copy(wbuf, accsp.at[pl.ds(s * RT + q * 128, 128)])

    def chunk1(i, _):
        b = w * EWP + i * CE
        pltpu.sync_copy(rows_hbm.at[pl.ds(b, CE)], rowc)
        for h in range(HEAD):
            for m in range(CE // 16):
                idxh[h, pl.ds(m * 16, 16)] = (
                    rowc[pl.ds(m * 16, 16)] + h * NP)
        for h in range(HEAD):
            pltpu.sync_copy(exp_hbm.at[pl.ds(h * EP + b, CE)], expb)
            d0 = pltpu.async_copy(np0_hbm.at[idxh.at[h]], nb0, sem)
            d1 = pltpu.async_copy(np1_hbm.at[idxh.at[h]], nb1, sem)
            d0.wait()
            d1.wait()
            for m in range(CE // 16):
                sl = pl.ds(m * 16, 16)
                attb[sl] = expb[sl] / (nb0[sl] + nb1[sl] + 1e-8)
            pltpu.sync_copy(attb, att_hbm.at[pl.ds(h * EP + b, CE)])
        return 0
    lax.fori_loop(0, EWP // CE, chunk1, 0)

    plsc.subcore_barrier()

    def chunk2(i, _):
        b = s * ECT + i * CE
        pltpu.sync_copy(rows_hbm.at[pl.ds(b, CE)], rowc)
        pltpu.sync_copy(cols_hbm.at[pl.ds(b, CE)], colc)
        for m in range(CE // 16):
            colc[pl.ds(m * 16, 16)] = colc[pl.ds(m * 16, 16)] + c * NP
        pltpu.async_copy(v_hbm.at[colc], vbuf, sem).wait()
        pltpu.sync_copy(exp_hbm.at[pl.ds((2 * c) * EP + b, CE)], m0b)
        pltpu.sync_copy(exp_hbm.at[pl.ds((2 * c + 1) * EP + b, CE)], m1b)

        def grp(g, _):
            s0 = _splats(m0b[pl.ds(g * 16, 16)], iot)
            s1 = _splats(m1b[pl.ds(g * 16, 16)], iot)
            for t in range(16):
                e = g * 16 + t
                for j in range(HF // 16):
                    bb = s0[t] if j < 4 else s1[t]
                    vbuf[e, pl.ds(j * 16, 16)] = (
                        vbuf[e, pl.ds(j * 16, 16)] * bb)
            return 0
        lax.fori_loop(0, CE // 16, grp, 0)
        pltpu.sync_copy(vbuf, accsp.at[rowc], add=True)
        return 0
    lax.fori_loop(0, ECT // CE, chunk2, 0)

    plsc.subcore_barrier()

    for q in range(RT // 128):
        r0 = s * RT + q * 128
        pltpu.sync_copy(accsp.at[pl.ds(r0, 128)], wbuf)
        pltpu.sync_copy(np0_hbm.at[pl.ds((2 * c) * NP + r0, 128)], nh0)
        pltpu.sync_copy(np1_hbm.at[pl.ds((2 * c) * NP + r0, 128)], ng0)
        pltpu.sync_copy(np0_hbm.at[pl.ds((2 * c + 1) * NP + r0, 128)], nh1)
        pltpu.sync_copy(np1_hbm.at[pl.ds((2 * c + 1) * NP + r0, 128)], ng1)
        for m in range(8):
            sl = pl.ds(m * 16, 16)
            nh0[sl] = 1.0 / (nh0[sl] + ng0[sl] + 1e-8)
            nh1[sl] = 1.0 / (nh1[sl] + ng1[sl] + 1e-8)

        def grp3(g, _):
            i0 = _splats(nh0[pl.ds(g * 16, 16)], iot)
            i1 = _splats(nh1[pl.ds(g * 16, 16)], iot)
            for t in range(16):
                r = g * 16 + t
                for j in range(HF // 16):
                    bb = i0[t] if j < 4 else i1[t]
                    wbuf[r, pl.ds(j * 16, 16)] = (
                        wbuf[r, pl.ds(j * 16, 16)] * bb)
            return 0
        lax.fori_loop(0, 8, grp3, 0)
        pltpu.sync_copy(wbuf, res_hbm.at[pl.ds(c * NP + r0, 128)])


_agg_pass = functools.partial(
    pl.kernel,
    out_type=[
        jax.ShapeDtypeStruct((HEAD * EP,), jnp.float32),
        jax.ShapeDtypeStruct((2 * NP, HF), jnp.float32),
    ],
    mesh=_mesh,
    scratch_types=[
        pltpu.VMEM((CE,), jnp.int32),
        pltpu.VMEM((CE,), jnp.int32),
        pltpu.VMEM((HEAD, CE), jnp.int32),
        pltpu.VMEM((CE,), jnp.float32),
        pltpu.VMEM((CE,), jnp.float32),
        pltpu.VMEM((CE,), jnp.float32),
        pltpu.VMEM((CE,), jnp.float32),
        pltpu.VMEM((CE,), jnp.float32),
        pltpu.VMEM((CE,), jnp.float32),
        pltpu.VMEM((CE, HF), jnp.float32),
        pltpu.VMEM((128, HF), jnp.float32),
        pltpu.VMEM((128,), jnp.float32),
        pltpu.VMEM((128,), jnp.float32),
        pltpu.VMEM((128,), jnp.float32),
        pltpu.VMEM((128,), jnp.float32),
        pltpu.VMEM_SHARED((NP, HF), jnp.float32),
        pltpu.SemaphoreType.DMA,
    ],
)(_agg_body)


def kernel(adj_indices, embeds, qTrans, kTrans, vTrans):
    rows = adj_indices[0]
    cols = adj_indices[1]
    pad_e = EP - E
    rows_p = jnp.concatenate(
        [rows, jnp.full((pad_e,), N + 200, jnp.int32)])
    cols_p = jnp.concatenate(
        [cols, jnp.full((pad_e,), N + 200, jnp.int32)])
    embeds_p = jnp.pad(embeds, ((0, NP - N), (0, 0)))
    q, k, vs = _qkv(embeds_p, qTrans, kTrans, vTrans)
    vflat = vs.reshape(2 * NP, HF)
    exp_flat, np0, np1 = _edge_pass(rows_p, cols_p, q, k)
    att_flat, res2 = _agg_pass(rows_p, cols_p, exp_flat, np0, np1, vflat)
    res = jnp.concatenate([res2[:N], res2[NP:NP + N]], axis=1)
    att = att_flat.reshape(HEAD, EP)[:, :E].T
    return res, att


# speedup vs baseline: 1.9389x; 1.0427x over previous
import functools

import jax
import jax.numpy as jnp
from jax import lax
from jax.experimental import pallas as pl
from jax.experimental.pallas import tpu as pltpu
from jax.experimental.pallas import tpu_sc as plsc

LATDIM = 256
HEAD = 4
DH = LATDIM // HEAD
N = 10000
E = 160000
NP = 10240
NF = NP * HEAD
NC = 2
NS = 16
NW = NC * NS
CE = 128
EWP = 5120
EP = EWP * NW
ECT = EP // NS
SL = NF // NS
HF = LATDIM // 2
RT = NP // NS

_mesh = plsc.VectorSubcoreMesh(core_axis_name="c", subcore_axis_name="s")


def _qkv_body(x_ref, q_ref, k_ref, v_ref, oq_ref, ok_ref, ov_ref):
    x = x_ref[...]
    oq_ref[...] = jnp.dot(x, q_ref[...], preferred_element_type=jnp.float32)
    ok_ref[...] = jnp.dot(x, k_ref[...], preferred_element_type=jnp.float32)
    v = jnp.dot(x, v_ref[...], preferred_element_type=jnp.float32)
    ov_ref[0] = v[:, :HF]
    ov_ref[1] = v[:, HF:]


_qkv = pl.pallas_call(
    _qkv_body,
    grid=(NP // 1024,),
    in_specs=[
        pl.BlockSpec((1024, LATDIM), lambda i: (i, 0)),
        pl.BlockSpec((LATDIM, LATDIM), lambda i: (0, 0)),
        pl.BlockSpec((LATDIM, LATDIM), lambda i: (0, 0)),
        pl.BlockSpec((LATDIM, LATDIM), lambda i: (0, 0)),
    ],
    out_specs=[
        pl.BlockSpec((1024, LATDIM), lambda i: (i, 0)),
        pl.BlockSpec((1024, LATDIM), lambda i: (i, 0)),
        pl.BlockSpec((2, 1024, HF), lambda i: (0, i, 0)),
    ],
    out_shape=[
        jax.ShapeDtypeStruct((NP, LATDIM), jnp.float32),
        jax.ShapeDtypeStruct((NP, LATDIM), jnp.float32),
        jax.ShapeDtypeStruct((2, NP, HF), jnp.float32),
    ],
)


def _lane_total(acc, iot):
    for bit in (1, 2, 4, 8):
        acc = acc + jnp.take(acc, iot ^ bit, mode='fill')
    return acc


def _splats(v, iot):
    vs = [v]
    for bit in (8, 4, 2, 1):
        nxt = []
        for u in vs:
            p = jnp.take(u, iot ^ bit, mode='fill')
            keep = (iot & bit) == 0
            nxt.append(jnp.where(keep, u, p))
            nxt.append(jnp.where(keep, p, u))
        vs = nxt
    return vs


def _edge_body(rows_hbm, cols_hbm, q_hbm, k_hbm,
               exp_hbm, np0_hbm, np1_hbm,
               rowc, colc, qbuf, kbuf, evbuf, idxh, zbuf, normsp, sem):
    c = lax.axis_index("c")
    s = lax.axis_index("s")
    w = s * NC + c
    iot = lax.iota(jnp.int32, 16)
    zero16 = jnp.zeros((16,), jnp.float32)

    def zb(j, _):
        zbuf[pl.ds(j * 16, 16)] = zero16
        return 0
    lax.fori_loop(0, SL // 16, zb, 0)
    pltpu.sync_copy(zbuf, normsp.at[pl.ds(s * SL, SL)])
    plsc.subcore_barrier()

    def chunk(i, _):
        b = w * EWP + i * CE
        pltpu.sync_copy(rows_hbm.at[pl.ds(b, CE)], rowc)
        pltpu.sync_copy(cols_hbm.at[pl.ds(b, CE)], colc)
        dq = pltpu.async_copy(q_hbm.at[rowc], qbuf, sem)
        dk = pltpu.async_copy(k_hbm.at[colc], kbuf, sem)
        dq.wait()
        dk.wait()
        for h in range(HEAD):
            for m in range(CE // 16):
                idxh[h, pl.ds(m * 16, 16)] = (
                    rowc[pl.ds(m * 16, 16)] + h * NP)

        def grp(g, _):
            dph = [jnp.zeros((16,), jnp.float32) for _ in range(HEAD)]
            for t in range(16):
                e = g * 16 + t
                for h in range(HEAD):
                    acc = (qbuf[e, pl.ds(h * DH, 16)]
                           * kbuf[e, pl.ds(h * DH, 16)])
                    for j in range(1, 4):
                        o = h * DH + j * 16
                        acc = acc + (qbuf[e, pl.ds(o, 16)]
                                     * kbuf[e, pl.ds(o, 16)])
                    tot = _lane_total(acc, iot)
                    dph[h] = jnp.where(iot == t, tot, dph[h])
            for h in range(HEAD):
                v = jnp.minimum(jnp.maximum(dph[h], -10.0), 10.0)
                evbuf[h, pl.ds(g * 16, 16)] = jnp.exp(v)
            return 0
        lax.fori_loop(0, CE // 16, grp, 0)

        for h in range(HEAD):
            pltpu.sync_copy(evbuf.at[h], exp_hbm.at[pl.ds(h * EP + b, CE)])
            pltpu.sync_copy(evbuf.at[h], normsp.at[idxh.at[h]], add=True)
        return 0
    lax.fori_loop(0, EWP // CE, chunk, 0)

    plsc.subcore_barrier()

    @pl.when(c == 0)
    def _():
        pltpu.sync_copy(normsp.at[pl.ds(s * SL, SL)],
                        np0_hbm.at[pl.ds(s * SL, SL)])

    @pl.when(c == 1)
    def _():
        pltpu.sync_copy(normsp.at[pl.ds(s * SL, SL)],
                        np1_hbm.at[pl.ds(s * SL, SL)])


_edge_pass = functools.partial(
    pl.kernel,
    out_type=[
        jax.ShapeDtypeStruct((HEAD * EP,), jnp.float32),
        jax.ShapeDtypeStruct((NF,), jnp.float32),
        jax.ShapeDtypeStruct((NF,), jnp.float32),
    ],
    mesh=_mesh,
    scratch_types=[
        pltpu.VMEM((CE,), jnp.int32),
        pltpu.VMEM((CE,), jnp.int32),
        pltpu.VMEM((CE, LATDIM), jnp.float32),
        pltpu.VMEM((CE, LATDIM), jnp.float32),
        pltpu.VMEM((HEAD, CE), jnp.float32),
        pltpu.VMEM((HEAD, CE), jnp.int32),
        pltpu.VMEM((SL,), jnp.float32),
        pltpu.VMEM_SHARED((NF,), jnp.float32),
        pltpu.SemaphoreType.DMA,
    ],
)(_edge_body)


def _agg_body(rows_hbm, cols_hbm, exp_hbm, np0_hbm, np1_hbm, v_hbm,
              att_hbm, res_hbm,
              rowc, colc, idxh, expb, nb0, nb1, attb, m0b, m1b,
              vbuf, wbuf, nh0, nh1, ng0, ng1, accsp, sem):
    c = lax.axis_index("c")
    s = lax.axis_index("s")
    w = s * NC + c
    iot = lax.iota(jnp.int32, 16)
    zero16 = jnp.zeros((16,), jnp.float32)

    def zrow(r, _):
        for j in range(HF // 16):
            wbuf[r, pl.ds(j * 16, 16)] = zero16
        return 0
    lax.fori_loop(0, 128, zrow, 0)
    for q in range(RT // 128):
        pltpu.sync_copy(wbuf, accsp.at[pl.ds(s * RT + q * 128, 128)])

    def chunk1(i, _):
        b = w * EWP + i * CE
        pltpu.sync_copy(rows_hbm.at[pl.ds(b, CE)], rowc)
        for h in range(HEAD):
            for m in range(CE // 16):
                idxh[h, pl.ds(m * 16, 16)] = (
                    rowc[pl.ds(m * 16, 16)] + h * NP)
        for h in range(HEAD):
            pltpu.sync_copy(exp_hbm.at[pl.ds(h * EP + b, CE)], expb)
            d0 = pltpu.async_copy(np0_hbm.at[idxh.at[h]], nb0, sem)
            d1 = pltpu.async_copy(np1_hbm.at[idxh.at[h]], nb1, sem)
            d0.wait()
            d1.wait()
            for m in range(CE // 16):
                sl = pl.ds(m * 16, 16)
                attb[sl] = expb[sl] / (nb0[sl] + nb1[sl] + 1e-8)
            pltpu.sync_copy(attb, att_hbm.at[pl.ds(h * EP + b, CE)])
        return 0
    lax.fori_loop(0, EWP // CE, chunk1, 0)

    plsc.subcore_barrier()

    def chunk2(i, _):
        b = s * ECT + i * CE
        pltpu.sync_copy(rows_hbm.at[pl.ds(b, CE)], rowc)
        pltpu.sync_copy(cols_hbm.at[pl.ds(b, CE)], colc)
        for m in range(CE // 16):
            colc[pl.ds(m * 16, 16)] = colc[pl.ds(m * 16, 16)] + c * NP
        dv = pltpu.async_copy(v_hbm.at[colc], vbuf, sem)
        pltpu.sync_copy(exp_hbm.at[pl.ds((2 * c) * EP + b, CE)], m0b)
        pltpu.sync_copy(exp_hbm.at[pl.ds((2 * c + 1) * EP + b, CE)], m1b)
        dv.wait()

        def grp(g, _):
            s0 = _splats(m0b[pl.ds(g * 16, 16)], iot)
            s1 = _splats(m1b[pl.ds(g * 16, 16)], iot)
            for t in range(16):
                e = g * 16 + t
                for j in range(HF // 16):
                    bb = s0[t] if j < 4 else s1[t]
                    vbuf[e, pl.ds(j * 16, 16)] = (
                        vbuf[e, pl.ds(j * 16, 16)] * bb)
            return 0
        lax.fori_loop(0, CE // 16, grp, 0)
        pltpu.sync_copy(vbuf, accsp.at[rowc], add=True)
        return 0
    lax.fori_loop(0, ECT // CE, chunk2, 0)

    plsc.subcore_barrier()

    for q in range(RT // 128):
        r0 = s * RT + q * 128
        pltpu.sync_copy(accsp.at[pl.ds(r0, 128)], wbuf)
        pltpu.sync_copy(np0_hbm.at[pl.ds((2 * c) * NP + r0, 128)], nh0)
        pltpu.sync_copy(np1_hbm.at[pl.ds((2 * c) * NP + r0, 128)], ng0)
        pltpu.sync_copy(np0_hbm.at[pl.ds((2 * c + 1) * NP + r0, 128)], nh1)
        pltpu.sync_copy(np1_hbm.at[pl.ds((2 * c + 1) * NP + r0, 128)], ng1)
        for m in range(8):
            sl = pl.ds(m * 16, 16)
            nh0[sl] = 1.0 / (nh0[sl] + ng0[sl] + 1e-8)
            nh1[sl] = 1.0 / (nh1[sl] + ng1[sl] + 1e-8)

        def grp3(g, _):
            i0 = _splats(nh0[pl.ds(g * 16, 16)], iot)
            i1 = _splats(nh1[pl.ds(g * 16, 16)], iot)
            for t in range(16):
                r = g * 16 + t
                for j in range(HF // 16):
                    bb = i0[t] if j < 4 else i1[t]
                    wbuf[r, pl.ds(j * 16, 16)] = (
                        wbuf[r, pl.ds(j * 16, 16)] * bb)
            return 0
        lax.fori_loop(0, 8, grp3, 0)
        pltpu.sync_copy(wbuf, res_hbm.at[pl.ds(c * NP + r0, 128)])


_agg_pass = functools.partial(
    pl.kernel,
    out_type=[
        jax.ShapeDtypeStruct((HEAD * EP,), jnp.float32),
        jax.ShapeDtypeStruct((2 * NP, HF), jnp.float32),
    ],
    mesh=_mesh,
    scratch_types=[
        pltpu.VMEM((CE,), jnp.int32),
        pltpu.VMEM((CE,), jnp.int32),
        pltpu.VMEM((HEAD, CE), jnp.int32),
        pltpu.VMEM((CE,), jnp.float32),
        pltpu.VMEM((CE,), jnp.float32),
        pltpu.VMEM((CE,), jnp.float32),
        pltpu.VMEM((CE,), jnp.float32),
        pltpu.VMEM((CE,), jnp.float32),
        pltpu.VMEM((CE,), jnp.float32),
        pltpu.VMEM((CE, HF), jnp.float32),
        pltpu.VMEM((128, HF), jnp.float32),
        pltpu.VMEM((128,), jnp.float32),
        pltpu.VMEM((128,), jnp.float32),
        pltpu.VMEM((128,), jnp.float32),
        pltpu.VMEM((128,), jnp.float32),
        pltpu.VMEM_SHARED((NP, HF), jnp.float32),
        pltpu.SemaphoreType.DMA,
    ],
)(_agg_body)


def kernel(adj_indices, embeds, qTrans, kTrans, vTrans):
    rows = adj_indices[0]
    cols = adj_indices[1]
    pad_e = EP - E
    rows_p = jnp.concatenate(
        [rows, jnp.full((pad_e,), N + 200, jnp.int32)])
    cols_p = jnp.concatenate(
        [cols, jnp.full((pad_e,), N + 200, jnp.int32)])
    embeds_p = jnp.pad(embeds, ((0, NP - N), (0, 0)))
    q, k, vs = _qkv(embeds_p, qTrans, kTrans, vTrans)
    vflat = vs.reshape(2 * NP, HF)
    exp_flat, np0, np1 = _edge_pass(rows_p, cols_p, q, k)
    att_flat, res2 = _agg_pass(rows_p, cols_p, exp_flat, np0, np1, vflat)
    res = jnp.concatenate([res2[:N], res2[NP:NP + N]], axis=1)
    att = att_flat.reshape(HEAD, EP)[:, :E].T
    return res, att


# speedup vs baseline: 2.3614x; 1.2179x over previous
import functools

import jax
import jax.numpy as jnp
from jax import lax
from jax.experimental import pallas as pl
from jax.experimental.pallas import tpu as pltpu
from jax.experimental.pallas import tpu_sc as plsc

LATDIM = 256
HEAD = 4
DH = LATDIM // HEAD
N = 10000
E = 160000
NP = 10240
NF = NP * HEAD
NC = 2
NS = 16
NW = NC * NS
CE = 128
EWP = 5120
EP = EWP * NW
ECT = EP // NS
SL = NF // NS
HF = LATDIM // 2
RT = NP // NS

_mesh = plsc.VectorSubcoreMesh(core_axis_name="c", subcore_axis_name="s")


def _qkv_body(x_ref, q_ref, k_ref, v_ref, oq_ref, ok_ref, ov_ref):
    x = x_ref[...]
    oq_ref[...] = jnp.dot(x, q_ref[...], preferred_element_type=jnp.float32)
    ok_ref[...] = jnp.dot(x, k_ref[...], preferred_element_type=jnp.float32)
    v = jnp.dot(x, v_ref[...], preferred_element_type=jnp.float32)
    ov_ref[0] = v[:, :HF]
    ov_ref[1] = v[:, HF:]


_qkv = pl.pallas_call(
    _qkv_body,
    grid=(NP // 1024,),
    in_specs=[
        pl.BlockSpec((1024, LATDIM), lambda i: (i, 0)),
        pl.BlockSpec((LATDIM, LATDIM), lambda i: (0, 0)),
        pl.BlockSpec((LATDIM, LATDIM), lambda i: (0, 0)),
        pl.BlockSpec((LATDIM, LATDIM), lambda i: (0, 0)),
    ],
    out_specs=[
        pl.BlockSpec((1024, LATDIM), lambda i: (i, 0)),
        pl.BlockSpec((1024, LATDIM), lambda i: (i, 0)),
        pl.BlockSpec((2, 1024, HF), lambda i: (0, i, 0)),
    ],
    out_shape=[
        jax.ShapeDtypeStruct((NP, LATDIM), jnp.float32),
        jax.ShapeDtypeStruct((NP, LATDIM), jnp.float32),
        jax.ShapeDtypeStruct((2, NP, HF), jnp.float32),
    ],
)


def _lane_total(acc, iot):
    for bit in (1, 2, 4, 8):
        acc = acc + jnp.take(acc, iot ^ bit, mode='fill')
    return acc


def _splats(v, iot):
    vs = [v]
    for bit in (8, 4, 2, 1):
        nxt = []
        for u in vs:
            p = jnp.take(u, iot ^ bit, mode='fill')
            keep = (iot & bit) == 0
            nxt.append(jnp.where(keep, u, p))
            nxt.append(jnp.where(keep, p, u))
        vs = nxt
    return vs


CA = 64
NCH = EWP // CA


def _edge_body(rows_hbm, cols_hbm, q_hbm, k_hbm,
               exp_hbm, np0_hbm, np1_hbm,
               rc0, cc0, rc1, cc1, qb0, kb0, qb1, kb1,
               evbuf, idxh, zbuf, normsp, sq0, sk0, sq1, sk1):
    c = lax.axis_index("c")
    s = lax.axis_index("s")
    w = s * NC + c
    iot = lax.iota(jnp.int32, 16)

    zero16 = jnp.zeros((16,), jnp.float32)

    def zb(j, _):
        zbuf[pl.ds(j * 16, 16)] = zero16
        return 0
    lax.fori_loop(0, SL // 16, zb, 0)
    pltpu.sync_copy(zbuf, normsp.at[pl.ds(s * SL, SL)])
    plsc.subcore_barrier()

    base = w * EWP

    def compute(qbuf, kbuf, rowc, b):
        for h in range(HEAD):
            for m in range(CA // 16):
                idxh[h, pl.ds(m * 16, 16)] = (
                    rowc[pl.ds(m * 16, 16)] + h * NP)

        def grp(g, _):
            dph = [jnp.zeros((16,), jnp.float32) for _ in range(HEAD)]
            for t in range(16):
                e = g * 16 + t
                for h in range(HEAD):
                    acc = (qbuf[e, pl.ds(h * DH, 16)]
                           * kbuf[e, pl.ds(h * DH, 16)])
                    for j in range(1, 4):
                        o = h * DH + j * 16
                        acc = acc + (qbuf[e, pl.ds(o, 16)]
                                     * kbuf[e, pl.ds(o, 16)])
                    tot = _lane_total(acc, iot)
                    dph[h] = jnp.where(iot == t, tot, dph[h])
            for h in range(HEAD):
                v = jnp.minimum(jnp.maximum(dph[h], -10.0), 10.0)
                evbuf[h, pl.ds(g * 16, 16)] = jnp.exp(v)
            return 0
        lax.fori_loop(0, CA // 16, grp, 0)

        for h in range(HEAD):
            pltpu.sync_copy(evbuf.at[h], exp_hbm.at[pl.ds(h * EP + b, CA)])
            pltpu.sync_copy(evbuf.at[h], normsp.at[idxh.at[h]], add=True)

    pltpu.sync_copy(rows_hbm.at[pl.ds(base, CA)], rc0)
    pltpu.sync_copy(cols_hbm.at[pl.ds(base, CA)], cc0)
    pltpu.async_copy(q_hbm.at[rc0], qb0, sq0)
    pltpu.async_copy(k_hbm.at[cc0], kb0, sk0)

    def body(i, _):
        b0 = base + (2 * i) * CA
        b1 = b0 + CA
        pltpu.sync_copy(rows_hbm.at[pl.ds(b1, CA)], rc1)
        pltpu.sync_copy(cols_hbm.at[pl.ds(b1, CA)], cc1)
        pltpu.async_copy(q_hbm.at[rc1], qb1, sq1)
        pltpu.async_copy(k_hbm.at[cc1], kb1, sk1)
        pltpu.make_async_copy(q_hbm.at[rc0], qb0, sq0).wait()
        pltpu.make_async_copy(k_hbm.at[cc0], kb0, sk0).wait()
        compute(qb0, kb0, rc0, b0)

        @pl.when(i < NCH // 2 - 1)
        def _():
            b2 = b0 + 2 * CA
            pltpu.sync_copy(rows_hbm.at[pl.ds(b2, CA)], rc0)
            pltpu.sync_copy(cols_hbm.at[pl.ds(b2, CA)], cc0)
            pltpu.async_copy(q_hbm.at[rc0], qb0, sq0)
            pltpu.async_copy(k_hbm.at[cc0], kb0, sk0)

        pltpu.make_async_copy(q_hbm.at[rc1], qb1, sq1).wait()
        pltpu.make_async_copy(k_hbm.at[cc1], kb1, sk1).wait()
        compute(qb1, kb1, rc1, b1)
        return 0
    lax.fori_loop(0, NCH // 2, body, 0)

    plsc.subcore_barrier()

    @pl.when(c == 0)
    def _():
        pltpu.sync_copy(normsp.at[pl.ds(s * SL, SL)],
                        np0_hbm.at[pl.ds(s * SL, SL)])

    @pl.when(c == 1)
    def _():
        pltpu.sync_copy(normsp.at[pl.ds(s * SL, SL)],
                        np1_hbm.at[pl.ds(s * SL, SL)])


_edge_pass = functools.partial(
    pl.kernel,
    out_type=[
        jax.ShapeDtypeStruct((HEAD * EP,), jnp.float32),
        jax.ShapeDtypeStruct((NF,), jnp.float32),
        jax.ShapeDtypeStruct((NF,), jnp.float32),
    ],
    mesh=_mesh,
    scratch_types=[
        pltpu.VMEM((CA,), jnp.int32),
        pltpu.VMEM((CA,), jnp.int32),
        pltpu.VMEM((CA,), jnp.int32),
        pltpu.VMEM((CA,), jnp.int32),
        pltpu.VMEM((CA, LATDIM), jnp.float32),
        pltpu.VMEM((CA, LATDIM), jnp.float32),
        pltpu.VMEM((CA, LATDIM), jnp.float32),
        pltpu.VMEM((CA, LATDIM), jnp.float32),
        pltpu.VMEM((HEAD, CA), jnp.float32),
        pltpu.VMEM((HEAD, CA), jnp.int32),
        pltpu.VMEM((SL,), jnp.float32),
        pltpu.VMEM_SHARED((NF,), jnp.float32),
        pltpu.SemaphoreType.DMA,
        pltpu.SemaphoreType.DMA,
        pltpu.SemaphoreType.DMA,
        pltpu.SemaphoreType.DMA,
    ],
)(_edge_body)


def _agg_body(rows_hbm, cols_hbm, exp_hbm, np0_hbm, np1_hbm, v_hbm,
              att_hbm, res_hbm,
              rowc, colc, idxh, expb, nb0, nb1, attb, m0b, m1b,
              vbuf, wbuf, nh0, nh1, ng0, ng1, accsp, sem):
    c = lax.axis_index("c")
    s = lax.axis_index("s")
    w = s * NC + c
    iot = lax.iota(jnp.int32, 16)
    zero16 = jnp.zeros((16,), jnp.float32)

    def zrow(r, _):
        for j in range(HF // 16):
            wbuf[r, pl.ds(j * 16, 16)] = zero16
        return 0
    lax.fori_loop(0, 128, zrow, 0)
    for q in range(RT // 128):
        pltpu.sync_copy(wbuf, accsp.at[pl.ds(s * RT + q * 128, 128)])

    def chunk1(i, _):
        b = w * EWP + i * CE
        pltpu.sync_copy(rows_hbm.at[pl.ds(b, CE)], rowc)
        for h in range(HEAD):
            for m in range(CE // 16):
                idxh[h, pl.ds(m * 16, 16)] = (
                    rowc[pl.ds(m * 16, 16)] + h * NP)
        for h in range(HEAD):
            pltpu.sync_copy(exp_hbm.at[pl.ds(h * EP + b, CE)], expb)
            d0 = pltpu.async_copy(np0_hbm.at[idxh.at[h]], nb0, sem)
            d1 = pltpu.async_copy(np1_hbm.at[idxh.at[h]], nb1, sem)
            d0.wait()
            d1.wait()
            for m in range(CE // 16):
                sl = pl.ds(m * 16, 16)
                attb[sl] = expb[sl] / (nb0[sl] + nb1[sl] + 1e-8)
            pltpu.sync_copy(attb, att_hbm.at[pl.ds(h * EP + b, CE)])
        return 0
    lax.fori_loop(0, EWP // CE, chunk1, 0)

    plsc.subcore_barrier()

    def chunk2(i, _):
        b = s * ECT + i * CE
        pltpu.sync_copy(rows_hbm.at[pl.ds(b, CE)], rowc)
        pltpu.sync_copy(cols_hbm.at[pl.ds(b, CE)], colc)
        for m in range(CE // 16):
            colc[pl.ds(m * 16, 16)] = colc[pl.ds(m * 16, 16)] + c * NP
        dv = pltpu.async_copy(v_hbm.at[colc], vbuf, sem)
        pltpu.sync_copy(exp_hbm.at[pl.ds((2 * c) * EP + b, CE)], m0b)
        pltpu.sync_copy(exp_hbm.at[pl.ds((2 * c + 1) * EP + b, CE)], m1b)
        dv.wait()

        def grp(g, _):
            s0 = _splats(m0b[pl.ds(g * 16, 16)], iot)
            s1 = _splats(m1b[pl.ds(g * 16, 16)], iot)
            for t in range(16):
                e = g * 16 + t
                for j in range(HF // 16):
                    bb = s0[t] if j < 4 else s1[t]
                    vbuf[e, pl.ds(j * 16, 16)] = (
                        vbuf[e, pl.ds(j * 16, 16)] * bb)
            return 0
        lax.fori_loop(0, CE // 16, grp, 0)
        pltpu.sync_copy(vbuf, accsp.at[rowc], add=True)
        return 0
    lax.fori_loop(0, ECT // CE, chunk2, 0)

    plsc.subcore_barrier()

    for q in range(RT // 128):
        r0 = s * RT + q * 128
        pltpu.sync_copy(accsp.at[pl.ds(r0, 128)], wbuf)
        pltpu.sync_copy(np0_hbm.at[pl.ds((2 * c) * NP + r0, 128)], nh0)
        pltpu.sync_copy(np1_hbm.at[pl.ds((2 * c) * NP + r0, 128)], ng0)
        pltpu.sync_copy(np0_hbm.at[pl.ds((2 * c + 1) * NP + r0, 128)], nh1)
        pltpu.sync_copy(np1_hbm.at[pl.ds((2 * c + 1) * NP + r0, 128)], ng1)
        for m in range(8):
            sl = pl.ds(m * 16, 16)
            nh0[sl] = 1.0 / (nh0[sl] + ng0[sl] + 1e-8)
            nh1[sl] = 1.0 / (nh1[sl] + ng1[sl] + 1e-8)

        def grp3(g, _):
            i0 = _splats(nh0[pl.ds(g * 16, 16)], iot)
            i1 = _splats(nh1[pl.ds(g * 16, 16)], iot)
            for t in range(16):
                r = g * 16 + t
                for j in range(HF // 16):
                    bb = i0[t] if j < 4 else i1[t]
                    wbuf[r, pl.ds(j * 16, 16)] = (
                        wbuf[r, pl.ds(j * 16, 16)] * bb)
            return 0
        lax.fori_loop(0, 8, grp3, 0)
        pltpu.sync_copy(wbuf, res_hbm.at[pl.ds(c * NP + r0, 128)])


_agg_pass = functools.partial(
    pl.kernel,
    out_type=[
        jax.ShapeDtypeStruct((HEAD * EP,), jnp.float32),
        jax.ShapeDtypeStruct((2 * NP, HF), jnp.float32),
    ],
    mesh=_mesh,
    scratch_types=[
        pltpu.VMEM((CE,), jnp.int32),
        pltpu.VMEM((CE,), jnp.int32),
        pltpu.VMEM((HEAD, CE), jnp.int32),
        pltpu.VMEM((CE,), jnp.float32),
        pltpu.VMEM((CE,), jnp.float32),
        pltpu.VMEM((CE,), jnp.float32),
        pltpu.VMEM((CE,), jnp.float32),
        pltpu.VMEM((CE,), jnp.float32),
        pltpu.VMEM((CE,), jnp.float32),
        pltpu.VMEM((CE, HF), jnp.float32),
        pltpu.VMEM((128, HF), jnp.float32),
        pltpu.VMEM((128,), jnp.float32),
        pltpu.VMEM((128,), jnp.float32),
        pltpu.VMEM((128,), jnp.float32),
        pltpu.VMEM((128,), jnp.float32),
        pltpu.VMEM_SHARED((NP, HF), jnp.float32),
        pltpu.SemaphoreType.DMA,
    ],
)(_agg_body)


def kernel(adj_indices, embeds, qTrans, kTrans, vTrans):
    rows = adj_indices[0]
    cols = adj_indices[1]
    pad_e = EP - E
    rows_p = jnp.concatenate(
        [rows, jnp.full((pad_e,), N + 200, jnp.int32)])
    cols_p = jnp.concatenate(
        [cols, jnp.full((pad_e,), N + 200, jnp.int32)])
    embeds_p = jnp.pad(embeds, ((0, NP - N), (0, 0)))
    q, k, vs = _qkv(embeds_p, qTrans, kTrans, vTrans)
    vflat = vs.reshape(2 * NP, HF)
    exp_flat, np0, np1 = _edge_pass(rows_p, cols_p, q, k)
    att_flat, res2 = _agg_pass(rows_p, cols_p, exp_flat, np0, np1, vflat)
    res = jnp.concatenate([res2[:N], res2[NP:NP + N]], axis=1)
    att = att_flat.reshape(HEAD, EP)[:, :E].T
    return res, att


# speedup vs baseline: 2.5462x; 1.0782x over previous
import functools

import jax
import jax.numpy as jnp
from jax import lax
from jax.experimental import pallas as pl
from jax.experimental.pallas import tpu as pltpu
from jax.experimental.pallas import tpu_sc as plsc

LATDIM = 256
HEAD = 4
DH = LATDIM // HEAD
N = 10000
E = 160000
NP = 10240
NF = NP * HEAD
NC = 2
NS = 16
NW = NC * NS
CE = 128
EWP = 5120
EP = EWP * NW
ECT = EP // NS
SL = NF // NS
HF = LATDIM // 2
RT = NP // NS

_mesh = plsc.VectorSubcoreMesh(core_axis_name="c", subcore_axis_name="s")


def _qkv_body(x_ref, q_ref, k_ref, v_ref, oq_ref, ok_ref, ov_ref):
    x = x_ref[...]
    oq_ref[...] = jnp.dot(x, q_ref[...], preferred_element_type=jnp.float32)
    ok_ref[...] = jnp.dot(x, k_ref[...], preferred_element_type=jnp.float32)
    v = jnp.dot(x, v_ref[...], preferred_element_type=jnp.float32)
    ov_ref[0] = v[:, :HF]
    ov_ref[1] = v[:, HF:]


_qkv = pl.pallas_call(
    _qkv_body,
    grid=(NP // 1024,),
    in_specs=[
        pl.BlockSpec((1024, LATDIM), lambda i: (i, 0)),
        pl.BlockSpec((LATDIM, LATDIM), lambda i: (0, 0)),
        pl.BlockSpec((LATDIM, LATDIM), lambda i: (0, 0)),
        pl.BlockSpec((LATDIM, LATDIM), lambda i: (0, 0)),
    ],
    out_specs=[
        pl.BlockSpec((1024, LATDIM), lambda i: (i, 0)),
        pl.BlockSpec((1024, LATDIM), lambda i: (i, 0)),
        pl.BlockSpec((2, 1024, HF), lambda i: (0, i, 0)),
    ],
    out_shape=[
        jax.ShapeDtypeStruct((NP, LATDIM), jnp.float32),
        jax.ShapeDtypeStruct((NP, LATDIM), jnp.float32),
        jax.ShapeDtypeStruct((2, NP, HF), jnp.float32),
    ],
)


def _lane_total(acc, iot):
    for bit in (1, 2, 4, 8):
        acc = acc + jnp.take(acc, iot ^ bit, mode='fill')
    return acc


def _splats(v, iot):
    vs = [v]
    for bit in (8, 4, 2, 1):
        nxt = []
        for u in vs:
            p = jnp.take(u, iot ^ bit, mode='fill')
            keep = (iot & bit) == 0
            nxt.append(jnp.where(keep, u, p))
            nxt.append(jnp.where(keep, p, u))
        vs = nxt
    return vs


CA = 64
NCH = EWP // CA


def _edge_body(rows_hbm, cols_hbm, q_hbm, k_hbm,
               exp_hbm, np0_hbm, np1_hbm,
               rc0, cc0, rc1, cc1, qb0, kb0, qb1, kb1,
               evbuf, idxh, zbuf, normsp, sq0, sk0, sq1, sk1):
    c = lax.axis_index("c")
    s = lax.axis_index("s")
    w = s * NC + c
    iot = lax.iota(jnp.int32, 16)

    zero16 = jnp.zeros((16,), jnp.float32)

    def zb(j, _):
        zbuf[pl.ds(j * 16, 16)] = zero16
        return 0
    lax.fori_loop(0, SL // 16, zb, 0)
    pltpu.sync_copy(zbuf, normsp.at[pl.ds(s * SL, SL)])
    plsc.subcore_barrier()

    base = w * EWP

    def compute(qbuf, kbuf, rowc, b):
        for h in range(HEAD):
            for m in range(CA // 16):
                idxh[h, pl.ds(m * 16, 16)] = (
                    rowc[pl.ds(m * 16, 16)] + h * NP)

        def grp(g, _):
            dph = [jnp.zeros((16,), jnp.float32) for _ in range(HEAD)]
            for t in range(16):
                e = g * 16 + t
                for h in range(HEAD):
                    acc = (qbuf[e, pl.ds(h * DH, 16)]
                           * kbuf[e, pl.ds(h * DH, 16)])
                    for j in range(1, 4):
                        o = h * DH + j * 16
                        acc = acc + (qbuf[e, pl.ds(o, 16)]
                                     * kbuf[e, pl.ds(o, 16)])
                    tot = _lane_total(acc, iot)
                    dph[h] = jnp.where(iot == t, tot, dph[h])
            for h in range(HEAD):
                v = jnp.minimum(jnp.maximum(dph[h], -10.0), 10.0)
                evbuf[h, pl.ds(g * 16, 16)] = jnp.exp(v)
            return 0
        lax.fori_loop(0, CA // 16, grp, 0)

        for h in range(HEAD):
            pltpu.sync_copy(evbuf.at[h], exp_hbm.at[pl.ds(h * EP + b, CA)])
            pltpu.sync_copy(evbuf.at[h], normsp.at[idxh.at[h]], add=True)

    pltpu.sync_copy(rows_hbm.at[pl.ds(base, CA)], rc0)
    pltpu.sync_copy(cols_hbm.at[pl.ds(base, CA)], cc0)
    pltpu.async_copy(q_hbm.at[rc0], qb0, sq0)
    pltpu.async_copy(k_hbm.at[cc0], kb0, sk0)

    def body(i, _):
        b0 = base + (2 * i) * CA
        b1 = b0 + CA
        pltpu.sync_copy(rows_hbm.at[pl.ds(b1, CA)], rc1)
        pltpu.sync_copy(cols_hbm.at[pl.ds(b1, CA)], cc1)
        pltpu.async_copy(q_hbm.at[rc1], qb1, sq1)
        pltpu.async_copy(k_hbm.at[cc1], kb1, sk1)
        pltpu.make_async_copy(q_hbm.at[rc0], qb0, sq0).wait()
        pltpu.make_async_copy(k_hbm.at[cc0], kb0, sk0).wait()
        compute(qb0, kb0, rc0, b0)

        @pl.when(i < NCH // 2 - 1)
        def _():
            b2 = b0 + 2 * CA
            pltpu.sync_copy(rows_hbm.at[pl.ds(b2, CA)], rc0)
            pltpu.sync_copy(cols_hbm.at[pl.ds(b2, CA)], cc0)
            pltpu.async_copy(q_hbm.at[rc0], qb0, sq0)
            pltpu.async_copy(k_hbm.at[cc0], kb0, sk0)

        pltpu.make_async_copy(q_hbm.at[rc1], qb1, sq1).wait()
        pltpu.make_async_copy(k_hbm.at[cc1], kb1, sk1).wait()
        compute(qb1, kb1, rc1, b1)
        return 0
    lax.fori_loop(0, NCH // 2, body, 0)

    plsc.subcore_barrier()

    @pl.when(c == 0)
    def _():
        pltpu.sync_copy(normsp.at[pl.ds(s * SL, SL)],
                        np0_hbm.at[pl.ds(s * SL, SL)])

    @pl.when(c == 1)
    def _():
        pltpu.sync_copy(normsp.at[pl.ds(s * SL, SL)],
                        np1_hbm.at[pl.ds(s * SL, SL)])


_edge_pass = functools.partial(
    pl.kernel,
    out_type=[
        jax.ShapeDtypeStruct((HEAD * EP,), jnp.float32),
        jax.ShapeDtypeStruct((NF,), jnp.float32),
        jax.ShapeDtypeStruct((NF,), jnp.float32),
    ],
    mesh=_mesh,
    scratch_types=[
        pltpu.VMEM((CA,), jnp.int32),
        pltpu.VMEM((CA,), jnp.int32),
        pltpu.VMEM((CA,), jnp.int32),
        pltpu.VMEM((CA,), jnp.int32),
        pltpu.VMEM((CA, LATDIM), jnp.float32),
        pltpu.VMEM((CA, LATDIM), jnp.float32),
        pltpu.VMEM((CA, LATDIM), jnp.float32),
        pltpu.VMEM((CA, LATDIM), jnp.float32),
        pltpu.VMEM((HEAD, CA), jnp.float32),
        pltpu.VMEM((HEAD, CA), jnp.int32),
        pltpu.VMEM((SL,), jnp.float32),
        pltpu.VMEM_SHARED((NF,), jnp.float32),
        pltpu.SemaphoreType.DMA,
        pltpu.SemaphoreType.DMA,
        pltpu.SemaphoreType.DMA,
        pltpu.SemaphoreType.DMA,
    ],
)(_edge_body)


def _agg_body(rows_hbm, cols_hbm, exp_hbm, np0_hbm, np1_hbm, v_hbm,
              att_hbm, res_hbm,
              rowc, colc, idxh, expb, nb0, nb1, attb, m0b, m1b,
              rc1, cc1, m2b, m3b, vb1,
              vbuf, nh0, nh1, ng0, ng1, accsp, sem, sem1):
    c = lax.axis_index("c")
    s = lax.axis_index("s")
    w = s * NC + c
    iot = lax.iota(jnp.int32, 16)
    zero16 = jnp.zeros((16,), jnp.float32)

    def zrow(r, _):
        for j in range(HF // 16):
            vbuf[r, pl.ds(j * 16, 16)] = zero16
        return 0
    lax.fori_loop(0, 128, zrow, 0)
    for q in range(RT // 128):
        pltpu.sync_copy(vbuf, accsp.at[pl.ds(s * RT + q * 128, 128)])

    def chunk1(i, _):
        b = w * EWP + i * CE
        pltpu.sync_copy(rows_hbm.at[pl.ds(b, CE)], rowc)
        for h in range(HEAD):
            for m in range(CE // 16):
                idxh[h, pl.ds(m * 16, 16)] = (
                    rowc[pl.ds(m * 16, 16)] + h * NP)
        for h in range(HEAD):
            pltpu.sync_copy(exp_hbm.at[pl.ds(h * EP + b, CE)], expb)
            d0 = pltpu.async_copy(np0_hbm.at[idxh.at[h]], nb0, sem)
            d1 = pltpu.async_copy(np1_hbm.at[idxh.at[h]], nb1, sem)
            d0.wait()
            d1.wait()
            for m in range(CE // 16):
                sl = pl.ds(m * 16, 16)
                attb[sl] = expb[sl] / (nb0[sl] + nb1[sl] + 1e-8)
            pltpu.sync_copy(attb, att_hbm.at[pl.ds(h * EP + b, CE)])
        return 0
    lax.fori_loop(0, EWP // CE, chunk1, 0)

    plsc.subcore_barrier()

    base2 = s * ECT

    def fire2(rcx, ccx, vbx, m0x, m1x, smx, b):
        pltpu.sync_copy(rows_hbm.at[pl.ds(b, CE)], rcx)
        pltpu.sync_copy(cols_hbm.at[pl.ds(b, CE)], ccx)
        for m in range(CE // 16):
            ccx[pl.ds(m * 16, 16)] = ccx[pl.ds(m * 16, 16)] + c * NP
        pltpu.async_copy(v_hbm.at[ccx], vbx, smx)
        pltpu.sync_copy(exp_hbm.at[pl.ds((2 * c) * EP + b, CE)], m0x)
        pltpu.sync_copy(exp_hbm.at[pl.ds((2 * c + 1) * EP + b, CE)], m1x)

    def scale2(rcx, ccx, vbx, m0x, m1x, smx):
        pltpu.make_async_copy(v_hbm.at[ccx], vbx, smx).wait()

        def grp(g, _):
            s0 = _splats(m0x[pl.ds(g * 16, 16)], iot)
            s1 = _splats(m1x[pl.ds(g * 16, 16)], iot)
            for t in range(16):
                e = g * 16 + t
                for j in range(HF // 16):
                    bb = s0[t] if j < 4 else s1[t]
                    vbx[e, pl.ds(j * 16, 16)] = (
                        vbx[e, pl.ds(j * 16, 16)] * bb)
            return 0
        lax.fori_loop(0, CE // 16, grp, 0)
        pltpu.sync_copy(vbx, accsp.at[rcx], add=True)

    fire2(rowc, colc, vbuf, m0b, m1b, sem, base2)

    def chunk2(i, _):
        b1 = base2 + (2 * i + 1) * CE
        fire2(rc1, cc1, vb1, m2b, m3b, sem1, b1)
        scale2(rowc, colc, vbuf, m0b, m1b, sem)

        @pl.when(i < ECT // CE // 2 - 1)
        def _():
            fire2(rowc, colc, vbuf, m0b, m1b, sem, b1 + CE)
        scale2(rc1, cc1, vb1, m2b, m3b, sem1)
        return 0
    lax.fori_loop(0, ECT // CE // 2, chunk2, 0)

    plsc.subcore_barrier()

    for q in range(RT // 128):
        r0 = s * RT + q * 128
        pltpu.sync_copy(accsp.at[pl.ds(r0, 128)], vbuf)
        pltpu.sync_copy(np0_hbm.at[pl.ds((2 * c) * NP + r0, 128)], nh0)
        pltpu.sync_copy(np1_hbm.at[pl.ds((2 * c) * NP + r0, 128)], ng0)
        pltpu.sync_copy(np0_hbm.at[pl.ds((2 * c + 1) * NP + r0, 128)], nh1)
        pltpu.sync_copy(np1_hbm.at[pl.ds((2 * c + 1) * NP + r0, 128)], ng1)
        for m in range(8):
            sl = pl.ds(m * 16, 16)
            nh0[sl] = 1.0 / (nh0[sl] + ng0[sl] + 1e-8)
            nh1[sl] = 1.0 / (nh1[sl] + ng1[sl] + 1e-8)

        def grp3(g, _):
            i0 = _splats(nh0[pl.ds(g * 16, 16)], iot)
            i1 = _splats(nh1[pl.ds(g * 16, 16)], iot)
            for t in range(16):
                r = g * 16 + t
                for j in range(HF // 16):
                    bb = i0[t] if j < 4 else i1[t]
                    vbuf[r, pl.ds(j * 16, 16)] = (
                        vbuf[r, pl.ds(j * 16, 16)] * bb)
            return 0
        lax.fori_loop(0, 8, grp3, 0)
        pltpu.sync_copy(vbuf, res_hbm.at[pl.ds(c * NP + r0, 128)])


_agg_pass = functools.partial(
    pl.kernel,
    out_type=[
        jax.ShapeDtypeStruct((HEAD * EP,), jnp.float32),
        jax.ShapeDtypeStruct((2 * NP, HF), jnp.float32),
    ],
    mesh=_mesh,
    scratch_types=[
        pltpu.VMEM((CE,), jnp.int32),
        pltpu.VMEM((CE,), jnp.int32),
        pltpu.VMEM((HEAD, CE), jnp.int32),
        pltpu.VMEM((CE,), jnp.float32),
        pltpu.VMEM((CE,), jnp.float32),
        pltpu.VMEM((CE,), jnp.float32),
        pltpu.VMEM((CE,), jnp.float32),
        pltpu.VMEM((CE,), jnp.float32),
        pltpu.VMEM((CE,), jnp.float32),
        pltpu.VMEM((CE,), jnp.int32),
        pltpu.VMEM((CE,), jnp.int32),
        pltpu.VMEM((CE,), jnp.float32),
        pltpu.VMEM((CE,), jnp.float32),
        pltpu.VMEM((CE, HF), jnp.float32),
        pltpu.VMEM((CE, HF), jnp.float32),
        pltpu.VMEM((128,), jnp.float32),
        pltpu.VMEM((128,), jnp.float32),
        pltpu.VMEM((128,), jnp.float32),
        pltpu.VMEM((128,), jnp.float32),
        pltpu.VMEM_SHARED((NP, HF), jnp.float32),
        pltpu.SemaphoreType.DMA,
        pltpu.SemaphoreType.DMA,
    ],
)(_agg_body)


def kernel(adj_indices, embeds, qTrans, kTrans, vTrans):
    rows = adj_indices[0]
    cols = adj_indices[1]
    pad_e = EP - E
    rows_p = jnp.concatenate(
        [rows, jnp.full((pad_e,), N + 200, jnp.int32)])
    cols_p = jnp.concatenate(
        [cols, jnp.full((pad_e,), N + 200, jnp.int32)])
    embeds_p = jnp.pad(embeds, ((0, NP - N), (0, 0)))
    q, k, vs = _qkv(embeds_p, qTrans, kTrans, vTrans)
    vflat = vs.reshape(2 * NP, HF)
    exp_flat, np0, np1 = _edge_pass(rows_p, cols_p, q, k)
    att_flat, res2 = _agg_pass(rows_p, cols_p, exp_flat, np0, np1, vflat)
    res = jnp.concatenate([res2[:N], res2[NP:NP + N]], axis=1)
    att = att_flat.reshape(HEAD, EP)[:, :E].T
    return res, att


# speedup vs baseline: 2.9424x; 1.1556x over previous
import functools

import jax
import jax.numpy as jnp
from jax import lax
from jax.experimental import pallas as pl
from jax.experimental.pallas import tpu as pltpu
from jax.experimental.pallas import tpu_sc as plsc

LATDIM = 256
HEAD = 4
DH = LATDIM // HEAD
N = 10000
E = 160000
NP = 10240
NF = NP * HEAD
NC = 2
NS = 16
NW = NC * NS
CE = 128
EWP = 5120
EP = EWP * NW
ECT = EP // NS
SL = NF // NS
HF = LATDIM // 2
RT = NP // NS

_mesh = plsc.VectorSubcoreMesh(core_axis_name="c", subcore_axis_name="s")


def _qkv_body(x_ref, q_ref, k_ref, v_ref, oq_ref, ok_ref, ov_ref):
    x = x_ref[...]
    oq_ref[...] = jnp.dot(x, q_ref[...], preferred_element_type=jnp.float32)
    ok_ref[...] = jnp.dot(x, k_ref[...], preferred_element_type=jnp.float32)
    v = jnp.dot(x, v_ref[...], preferred_element_type=jnp.float32)
    ov_ref[0] = v[:, :HF]
    ov_ref[1] = v[:, HF:]


_qkv = pl.pallas_call(
    _qkv_body,
    grid=(NP // 1024,),
    in_specs=[
        pl.BlockSpec((1024, LATDIM), lambda i: (i, 0)),
        pl.BlockSpec((LATDIM, LATDIM), lambda i: (0, 0)),
        pl.BlockSpec((LATDIM, LATDIM), lambda i: (0, 0)),
        pl.BlockSpec((LATDIM, LATDIM), lambda i: (0, 0)),
    ],
    out_specs=[
        pl.BlockSpec((1024, LATDIM), lambda i: (i, 0)),
        pl.BlockSpec((1024, LATDIM), lambda i: (i, 0)),
        pl.BlockSpec((2, 1024, HF), lambda i: (0, i, 0)),
    ],
    out_shape=[
        jax.ShapeDtypeStruct((NP, LATDIM), jnp.float32),
        jax.ShapeDtypeStruct((NP, LATDIM), jnp.float32),
        jax.ShapeDtypeStruct((2, NP, HF), jnp.float32),
    ],
)


def _lane_total(acc, iot):
    for bit in (1, 2, 4, 8):
        acc = acc + jnp.take(acc, iot ^ bit, mode='fill')
    return acc


def _splats(v, iot):
    vs = [v]
    for bit in (8, 4, 2, 1):
        nxt = []
        for u in vs:
            p = jnp.take(u, iot ^ bit, mode='fill')
            keep = (iot & bit) == 0
            nxt.append(jnp.where(keep, u, p))
            nxt.append(jnp.where(keep, p, u))
        vs = nxt
    return vs


CA = 64
NCH = EWP // CA


def _edge_body(rows_hbm, cols_hbm, q_hbm, k_hbm,
               exp_hbm, np0_hbm, np1_hbm,
               rc0, cc0, rc1, cc1, qb0, kb0, qb1, kb1,
               evbuf, idxh, zbuf, normsp, sq0, sk0, sq1, sk1):
    c = lax.axis_index("c")
    s = lax.axis_index("s")
    w = s * NC + c
    iot = lax.iota(jnp.int32, 16)

    zero16 = jnp.zeros((16,), jnp.float32)

    def zb(j, _):
        zbuf[pl.ds(j * 16, 16)] = zero16
        return 0
    lax.fori_loop(0, SL // 16, zb, 0)
    pltpu.sync_copy(zbuf, normsp.at[pl.ds(s * SL, SL)])
    plsc.subcore_barrier()

    base = w * EWP

    def compute(qbuf, kbuf, rowc, b):
        for h in range(HEAD):
            for m in range(CA // 16):
                idxh[h, pl.ds(m * 16, 16)] = (
                    rowc[pl.ds(m * 16, 16)] + h * NP)

        def grp(g, _):
            dph = [jnp.zeros((16,), jnp.float32) for _ in range(HEAD)]
            for t in range(16):
                e = g * 16 + t
                for h in range(HEAD):
                    acc = (qbuf[e, pl.ds(h * DH, 16)]
                           * kbuf[e, pl.ds(h * DH, 16)])
                    for j in range(1, 4):
                        o = h * DH + j * 16
                        acc = acc + (qbuf[e, pl.ds(o, 16)]
                                     * kbuf[e, pl.ds(o, 16)])
                    tot = _lane_total(acc, iot)
                    dph[h] = jnp.where(iot == t, tot, dph[h])
            for h in range(HEAD):
                v = jnp.minimum(jnp.maximum(dph[h], -10.0), 10.0)
                evbuf[h, pl.ds(g * 16, 16)] = jnp.exp(v)
            return 0
        lax.fori_loop(0, CA // 16, grp, 0)

        for h in range(HEAD):
            pltpu.sync_copy(evbuf.at[h], exp_hbm.at[pl.ds(h * EP + b, CA)])
            pltpu.sync_copy(evbuf.at[h], normsp.at[idxh.at[h]], add=True)

    pltpu.sync_copy(rows_hbm.at[pl.ds(base, CA)], rc0)
    pltpu.sync_copy(cols_hbm.at[pl.ds(base, CA)], cc0)
    pltpu.async_copy(q_hbm.at[rc0], qb0, sq0)
    pltpu.async_copy(k_hbm.at[cc0], kb0, sk0)

    def body(i, _):
        b0 = base + (2 * i) * CA
        b1 = b0 + CA
        pltpu.sync_copy(rows_hbm.at[pl.ds(b1, CA)], rc1)
        pltpu.sync_copy(cols_hbm.at[pl.ds(b1, CA)], cc1)
        pltpu.async_copy(q_hbm.at[rc1], qb1, sq1)
        pltpu.async_copy(k_hbm.at[cc1], kb1, sk1)
        pltpu.make_async_copy(q_hbm.at[rc0], qb0, sq0).wait()
        pltpu.make_async_copy(k_hbm.at[cc0], kb0, sk0).wait()
        compute(qb0, kb0, rc0, b0)

        @pl.when(i < NCH // 2 - 1)
        def _():
            b2 = b0 + 2 * CA
            pltpu.sync_copy(rows_hbm.at[pl.ds(b2, CA)], rc0)
            pltpu.sync_copy(cols_hbm.at[pl.ds(b2, CA)], cc0)
            pltpu.async_copy(q_hbm.at[rc0], qb0, sq0)
            pltpu.async_copy(k_hbm.at[cc0], kb0, sk0)

        pltpu.make_async_copy(q_hbm.at[rc1], qb1, sq1).wait()
        pltpu.make_async_copy(k_hbm.at[cc1], kb1, sk1).wait()
        compute(qb1, kb1, rc1, b1)
        return 0
    lax.fori_loop(0, NCH // 2, body, 0)

    plsc.subcore_barrier()

    @pl.when(c == 0)
    def _():
        pltpu.sync_copy(normsp.at[pl.ds(s * SL, SL)],
                        np0_hbm.at[pl.ds(s * SL, SL)])

    @pl.when(c == 1)
    def _():
        pltpu.sync_copy(normsp.at[pl.ds(s * SL, SL)],
                        np1_hbm.at[pl.ds(s * SL, SL)])


_edge_pass = functools.partial(
    pl.kernel,
    out_type=[
        jax.ShapeDtypeStruct((HEAD * EP,), jnp.float32),
        jax.ShapeDtypeStruct((NF,), jnp.float32),
        jax.ShapeDtypeStruct((NF,), jnp.float32),
    ],
    mesh=_mesh,
    scratch_types=[
        pltpu.VMEM((CA,), jnp.int32),
        pltpu.VMEM((CA,), jnp.int32),
        pltpu.VMEM((CA,), jnp.int32),
        pltpu.VMEM((CA,), jnp.int32),
        pltpu.VMEM((CA, LATDIM), jnp.float32),
        pltpu.VMEM((CA, LATDIM), jnp.float32),
        pltpu.VMEM((CA, LATDIM), jnp.float32),
        pltpu.VMEM((CA, LATDIM), jnp.float32),
        pltpu.VMEM((HEAD, CA), jnp.float32),
        pltpu.VMEM((HEAD, CA), jnp.int32),
        pltpu.VMEM((SL,), jnp.float32),
        pltpu.VMEM_SHARED((NF,), jnp.float32),
        pltpu.SemaphoreType.DMA,
        pltpu.SemaphoreType.DMA,
        pltpu.SemaphoreType.DMA,
        pltpu.SemaphoreType.DMA,
    ],
)(_edge_body)


def _agg_body(rows_hbm, cols_hbm, exp_hbm, np0_hbm, np1_hbm, v_hbm,
              att_hbm, res_hbm,
              rowc, colc, idxh, expb, nb0, nb1, attb, m0b, m1b,
              rc1, cc1, m2b, m3b, vb1,
              vbuf, nh0, nh1, ng0, ng1, accsp, sem, sem1):
    c = lax.axis_index("c")
    s = lax.axis_index("s")
    w = s * NC + c
    iot = lax.iota(jnp.int32, 16)
    zero16 = jnp.zeros((16,), jnp.float32)

    def zrow(r, _):
        for j in range(HF // 16):
            vbuf[r, pl.ds(j * 16, 16)] = zero16
        return 0
    lax.fori_loop(0, 128, zrow, 0)
    for q in range(RT // 128):
        pltpu.sync_copy(vbuf, accsp.at[pl.ds(s * RT + q * 128, 128)])

    def chunk1(i, _):
        b = w * EWP + i * CE
        pltpu.sync_copy(rows_hbm.at[pl.ds(b, CE)], rowc)
        for h in range(HEAD):
            for m in range(CE // 16):
                idxh[h, pl.ds(m * 16, 16)] = (
                    rowc[pl.ds(m * 16, 16)] + h * NP)
        for h in range(HEAD):
            pltpu.async_copy(
                exp_hbm.at[pl.ds(h * EP + b, CE)], expb.at[h], sem)
            pltpu.async_copy(np0_hbm.at[idxh.at[h]], nb0.at[h], sem)
            pltpu.async_copy(np1_hbm.at[idxh.at[h]], nb1.at[h], sem)
        for h in range(HEAD):
            pltpu.make_async_copy(
                exp_hbm.at[pl.ds(h * EP + b, CE)], expb.at[h], sem).wait()
            pltpu.make_async_copy(np0_hbm.at[idxh.at[h]], nb0.at[h], sem).wait()
            pltpu.make_async_copy(np1_hbm.at[idxh.at[h]], nb1.at[h], sem).wait()
        for h in range(HEAD):
            for m in range(CE // 16):
                sl = pl.ds(m * 16, 16)
                attb[h, sl] = expb[h, sl] / (nb0[h, sl] + nb1[h, sl] + 1e-8)
            pltpu.sync_copy(attb.at[h], att_hbm.at[pl.ds(h * EP + b, CE)])
        return 0
    lax.fori_loop(0, EWP // CE, chunk1, 0)

    plsc.subcore_barrier()

    base2 = s * ECT

    def fire2(rcx, ccx, vbx, m0x, m1x, smx, b):
        pltpu.sync_copy(rows_hbm.at[pl.ds(b, CE)], rcx)
        pltpu.sync_copy(cols_hbm.at[pl.ds(b, CE)], ccx)
        for m in range(CE // 16):
            ccx[pl.ds(m * 16, 16)] = ccx[pl.ds(m * 16, 16)] + c * NP
        pltpu.async_copy(v_hbm.at[ccx], vbx, smx)
        pltpu.sync_copy(exp_hbm.at[pl.ds((2 * c) * EP + b, CE)], m0x)
        pltpu.sync_copy(exp_hbm.at[pl.ds((2 * c + 1) * EP + b, CE)], m1x)

    def scale2(rcx, ccx, vbx, m0x, m1x, smx):
        pltpu.make_async_copy(v_hbm.at[ccx], vbx, smx).wait()

        def grp(g, _):
            s0 = _splats(m0x[pl.ds(g * 16, 16)], iot)
            s1 = _splats(m1x[pl.ds(g * 16, 16)], iot)
            for t in range(16):
                e = g * 16 + t
                for j in range(HF // 16):
                    bb = s0[t] if j < 4 else s1[t]
                    vbx[e, pl.ds(j * 16, 16)] = (
                        vbx[e, pl.ds(j * 16, 16)] * bb)
            return 0
        lax.fori_loop(0, CE // 16, grp, 0)
        pltpu.sync_copy(vbx, accsp.at[rcx], add=True)

    fire2(rowc, colc, vbuf, m0b, m1b, sem, base2)

    def chunk2(i, _):
        b1 = base2 + (2 * i + 1) * CE
        fire2(rc1, cc1, vb1, m2b, m3b, sem1, b1)
        scale2(rowc, colc, vbuf, m0b, m1b, sem)

        @pl.when(i < ECT // CE // 2 - 1)
        def _():
            fire2(rowc, colc, vbuf, m0b, m1b, sem, b1 + CE)
        scale2(rc1, cc1, vb1, m2b, m3b, sem1)
        return 0
    lax.fori_loop(0, ECT // CE // 2, chunk2, 0)

    plsc.subcore_barrier()

    for q in range(RT // 128):
        r0 = s * RT + q * 128
        pltpu.sync_copy(accsp.at[pl.ds(r0, 128)], vbuf)
        pltpu.sync_copy(np0_hbm.at[pl.ds((2 * c) * NP + r0, 128)], nh0)
        pltpu.sync_copy(np1_hbm.at[pl.ds((2 * c) * NP + r0, 128)], ng0)
        pltpu.sync_copy(np0_hbm.at[pl.ds((2 * c + 1) * NP + r0, 128)], nh1)
        pltpu.sync_copy(np1_hbm.at[pl.ds((2 * c + 1) * NP + r0, 128)], ng1)
        for m in range(8):
            sl = pl.ds(m * 16, 16)
            nh0[sl] = 1.0 / (nh0[sl] + ng0[sl] + 1e-8)
            nh1[sl] = 1.0 / (nh1[sl] + ng1[sl] + 1e-8)

        def grp3(g, _):
            i0 = _splats(nh0[pl.ds(g * 16, 16)], iot)
            i1 = _splats(nh1[pl.ds(g * 16, 16)], iot)
            for t in range(16):
                r = g * 16 + t
                for j in range(HF // 16):
                    bb = i0[t] if j < 4 else i1[t]
                    vbuf[r, pl.ds(j * 16, 16)] = (
                        vbuf[r, pl.ds(j * 16, 16)] * bb)
            return 0
        lax.fori_loop(0, 8, grp3, 0)
        pltpu.sync_copy(vbuf, res_hbm.at[pl.ds(c * NP + r0, 128)])


_agg_pass = functools.partial(
    pl.kernel,
    out_type=[
        jax.ShapeDtypeStruct((HEAD * EP,), jnp.float32),
        jax.ShapeDtypeStruct((2 * NP, HF), jnp.float32),
    ],
    mesh=_mesh,
    scratch_types=[
        pltpu.VMEM((CE,), jnp.int32),
        pltpu.VMEM((CE,), jnp.int32),
        pltpu.VMEM((HEAD, CE), jnp.int32),
        pltpu.VMEM((HEAD, CE), jnp.float32),
        pltpu.VMEM((HEAD, CE), jnp.float32),
        pltpu.VMEM((HEAD, CE), jnp.float32),
        pltpu.VMEM((HEAD, CE), jnp.float32),
        pltpu.VMEM((CE,), jnp.float32),
        pltpu.VMEM((CE,), jnp.float32),
        pltpu.VMEM((CE,), jnp.int32),
        pltpu.VMEM((CE,), jnp.int32),
        pltpu.VMEM((CE,), jnp.float32),
        pltpu.VMEM((CE,), jnp.float32),
        pltpu.VMEM((CE, HF), jnp.float32),
        pltpu.VMEM((CE, HF), jnp.float32),
        pltpu.VMEM((128,), jnp.float32),
        pltpu.VMEM((128,), jnp.float32),
        pltpu.VMEM((128,), jnp.float32),
        pltpu.VMEM((128,), jnp.float32),
        pltpu.VMEM_SHARED((NP, HF), jnp.float32),
        pltpu.SemaphoreType.DMA,
        pltpu.SemaphoreType.DMA,
    ],
)(_agg_body)


def kernel(adj_indices, embeds, qTrans, kTrans, vTrans):
    rows = adj_indices[0]
    cols = adj_indices[1]
    pad_e = EP - E
    rows_p = jnp.concatenate(
        [rows, jnp.full((pad_e,), N + 200, jnp.int32)])
    cols_p = jnp.concatenate(
        [cols, jnp.full((pad_e,), N + 200, jnp.int32)])
    embeds_p = jnp.pad(embeds, ((0, NP - N), (0, 0)))
    q, k, vs = _qkv(embeds_p, qTrans, kTrans, vTrans)
    vflat = vs.reshape(2 * NP, HF)
    exp_flat, np0, np1 = _edge_pass(rows_p, cols_p, q, k)
    att_flat, res2 = _agg_pass(rows_p, cols_p, exp_flat, np0, np1, vflat)
    res = jnp.concatenate([res2[:N], res2[NP:NP + N]], axis=1)
    att = att_flat.reshape(HEAD, EP)[:, :E].T
    return res, att
